# speedup vs baseline: 14.1927x; 14.1927x over previous
import functools

import jax
import jax.numpy as jnp
from jax import lax
from jax.experimental import pallas as pl
from jax.experimental.pallas import tpu as pltpu
from jax.experimental.pallas import tpu_sc as plsc

N = 10000
D = 128
NC, NS = 2, 16
NW = NC * NS
N_EXT = 10240
ROWS_PER_SUB = N_EXT // NS

E_REAL = 320000
CH = 128
EC = 80
E_PAD = NW * EC * CH

L_REAL = 100000
LC = 25
L_PAD = NW * LC * CH

_mesh = plsc.VectorSubcoreMesh(core_axis_name="c", subcore_axis_name="s")


def _sc_deg_body(dst_hbm, zeros1_hbm, degp_hbm, idx_v, ones_v, acc):
    c = lax.axis_index("c")
    s = lax.axis_index("s")
    w = c * NS + s
    for i in range(CH // 16):
        ones_v[pl.ds(i * 16, 16)] = jnp.full((16,), 1.0, jnp.float32)
    base = s * ROWS_PER_SUB
    pltpu.sync_copy(zeros1_hbm, acc.at[pl.ds(base, ROWS_PER_SUB)])
    plsc.subcore_barrier()

    def body(j, carry):
        pltpu.sync_copy(dst_hbm.at[w, j], idx_v)
        pltpu.sync_copy(ones_v, acc.at[idx_v], add=True)
        return carry

    lax.fori_loop(0, EC, body, 0)
    plsc.subcore_barrier()
    pltpu.sync_copy(acc.at[pl.ds(base, ROWS_PER_SUB)],
                    degp_hbm.at[c, pl.ds(base, ROWS_PER_SUB)])


_sc_deg = pl.kernel(
    _sc_deg_body,
    out_type=jax.ShapeDtypeStruct((NC, N_EXT), jnp.float32),
    mesh=_mesh,
    scratch_types=[
        pltpu.VMEM((CH,), jnp.int32),
        pltpu.VMEM((CH,), jnp.float32),
        pltpu.VMEM_SHARED((N_EXT,), jnp.float32),
    ],
)


def _sc_hop_body(t_hbm, src_hbm, dst_hbm, zeros_hbm, out_hbm,
                 sidx, didx, rows, acc, gsem):
    c = lax.axis_index("c")
    s = lax.axis_index("s")
    w = c * NS + s
    base = s * ROWS_PER_SUB

    @pl.when(c == 0)
    def _():
        pltpu.sync_copy(t_hbm.at[pl.ds(base, ROWS_PER_SUB)],
                        acc.at[pl.ds(base, ROWS_PER_SUB)])

    @pl.when(c != 0)
    def _():
        pltpu.sync_copy(zeros_hbm, acc.at[pl.ds(base, ROWS_PER_SUB)])

    pltpu.sync_copy(src_hbm.at[w], sidx)
    pltpu.sync_copy(dst_hbm.at[w], didx)
    plsc.subcore_barrier()

    def body(j, carry):
        pltpu.async_copy(t_hbm.at[sidx.at[j]], rows, gsem).wait()
        pltpu.sync_copy(rows, acc.at[didx.at[j]], add=True)
        return carry

    lax.fori_loop(0, EC, body, 0)
    plsc.subcore_barrier()
    pltpu.sync_copy(acc.at[pl.ds(base, ROWS_PER_SUB)],
                    out_hbm.at[c, pl.ds(base, ROWS_PER_SUB)])


_sc_hop = pl.kernel(
    _sc_hop_body,
    out_type=jax.ShapeDtypeStruct((NC, N_EXT, D), jnp.float32),
    mesh=_mesh,
    scratch_types=[
        pltpu.VMEM((EC, CH), jnp.int32),
        pltpu.VMEM((EC, CH), jnp.int32),
        pltpu.VMEM((CH, D), jnp.float32),
        pltpu.VMEM_SHARED((N_EXT, D), jnp.float32),
        pltpu.SemaphoreType.DMA,
    ],
)


def _sc_dec_body(z_hbm, i0_hbm, i1_hbm, s_hbm, i0v, i1v, ra, rb, sem0, sem1):
    c = lax.axis_index("c")
    s = lax.axis_index("s")
    w = c * NS + s
    pltpu.sync_copy(i0_hbm.at[w], i0v)
    pltpu.sync_copy(i1_hbm.at[w], i1v)

    def body(j, carry):
        cp0 = pltpu.async_copy(z_hbm.at[i0v.at[j]], ra, sem0)
        cp1 = pltpu.async_copy(z_hbm.at[i1v.at[j]], rb, sem1)
        cp0.wait()
        cp1.wait()

        def mul_row(r, carry2):
            for i in range(D // 16):
                sl = pl.ds(i * 16, 16)
                ra[r, sl] = ra[r, sl] * rb[r, sl]
            return carry2

        lax.fori_loop(0, CH, mul_row, 0)
        pltpu.sync_copy(ra, s_hbm.at[pl.ds(w * (LC * CH) + j * CH, CH)])
        return carry

    lax.fori_loop(0, LC, body, 0)


_sc_dec = pl.kernel(
    _sc_dec_body,
    out_type=jax.ShapeDtypeStruct((L_PAD, D), jnp.float32),
    mesh=_mesh,
    scratch_types=[
        pltpu.VMEM((LC, CH), jnp.int32),
        pltpu.VMEM((LC, CH), jnp.int32),
        pltpu.VMEM((CH, D), jnp.float32),
        pltpu.VMEM((CH, D), jnp.float32),
        pltpu.SemaphoreType.DMA,
        pltpu.SemaphoreType.DMA,
    ],
)


_BLK = 1024


def _tc_scale_x_body(degp_ref, x_ref, o_ref):
    deg = degp_ref[0] + degp_ref[1] + 1.0
    dinv = lax.rsqrt(deg)
    o_ref[...] = x_ref[...] * dinv[:, None]


def _tc_mid_body(degp_ref, up_ref, o_ref):
    deg = degp_ref[0] + degp_ref[1] + 1.0
    u = up_ref[0] + up_ref[1]
    o_ref[...] = u / deg[:, None]


def _tc_z_body(degp_ref, up_ref, w_ref, b_ref, o_ref):
    deg = degp_ref[0] + degp_ref[1] + 1.0
    dinv = lax.rsqrt(deg)
    h = (up_ref[0] + up_ref[1]) * dinv[:, None]
    o_ref[...] = (
        jnp.dot(h, w_ref[...], preferred_element_type=jnp.float32) + b_ref[...]
    )


def _tc_mlp_body(s_ref, w1_ref, b1_ref, w2_ref, b2_ref, o_ref):
    h = jnp.maximum(
        jnp.dot(s_ref[...], w1_ref[...], preferred_element_type=jnp.float32)
        + b1_ref[...], 0.0)
    o_ref[...] = (
        jnp.dot(h, w2_ref[...], preferred_element_type=jnp.float32)
        + b2_ref[...]
    )


def _tc_scale_x(degp, x_ext):
    grid = N_EXT // _BLK
    return pl.pallas_call(
        _tc_scale_x_body,
        grid=(grid,),
        in_specs=[
            pl.BlockSpec((NC, _BLK), lambda i: (0, i)),
            pl.BlockSpec((_BLK, D), lambda i: (i, 0)),
        ],
        out_specs=pl.BlockSpec((_BLK, D), lambda i: (i, 0)),
        out_shape=jax.ShapeDtypeStruct((N_EXT, D), jnp.float32),
    )(degp, x_ext)


def _tc_mid(degp, up):
    grid = N_EXT // _BLK
    return pl.pallas_call(
        _tc_mid_body,
        grid=(grid,),
        in_specs=[
            pl.BlockSpec((NC, _BLK), lambda i: (0, i)),
            pl.BlockSpec((NC, _BLK, D), lambda i: (0, i, 0)),
        ],
        out_specs=pl.BlockSpec((_BLK, D), lambda i: (i, 0)),
        out_shape=jax.ShapeDtypeStruct((N_EXT, D), jnp.float32),
    )(degp, up)


def _tc_z(degp, up, wT, b):
    grid = N_EXT // _BLK
    return pl.pallas_call(
        _tc_z_body,
        grid=(grid,),
        in_specs=[
            pl.BlockSpec((NC, _BLK), lambda i: (0, i)),
            pl.BlockSpec((NC, _BLK, D), lambda i: (0, i, 0)),
            pl.BlockSpec((D, D), lambda i: (0, 0)),
            pl.BlockSpec((1, D), lambda i: (0, 0)),
        ],
        out_specs=pl.BlockSpec((_BLK, D), lambda i: (i, 0)),
        out_shape=jax.ShapeDtypeStruct((N_EXT, D), jnp.float32),
    )(degp, up, wT, b)


def _tc_mlp(s, w1T, b1, w2T, b2):
    grid = L_PAD // _BLK
    return pl.pallas_call(
        _tc_mlp_body,
        grid=(grid,),
        in_specs=[
            pl.BlockSpec((_BLK, D), lambda i: (i, 0)),
            pl.BlockSpec((D, D // 2), lambda i: (0, 0)),
            pl.BlockSpec((1, D // 2), lambda i: (0, 0)),
            pl.BlockSpec((D // 2, 1), lambda i: (0, 0)),
            pl.BlockSpec((1, 1), lambda i: (0, 0)),
        ],
        out_specs=pl.BlockSpec((_BLK, 1), lambda i: (i, 0)),
        out_shape=jax.ShapeDtypeStruct((L_PAD, 1), jnp.float32),
    )(s, w1T, b1, w2T, b2)


@jax.jit
def kernel(x, edge_index, edge_label_index, W_conv, b_conv, W1, b1, W2, b2):
    src = edge_index[0].astype(jnp.int32)
    dst = edge_index[1].astype(jnp.int32)
    ep = E_PAD - E_REAL
    pad_gather = jnp.arange(ep, dtype=jnp.int32) % N
    pad_scatter = jnp.arange(ep, dtype=jnp.int32) % 16 + N
    src_p = jnp.concatenate([src, pad_gather]).reshape(NW, EC, CH)
    dst_p = jnp.concatenate([dst, pad_scatter]).reshape(NW, EC, CH)

    i0 = edge_label_index[0].astype(jnp.int32)
    i1 = edge_label_index[1].astype(jnp.int32)
    lp = L_PAD - L_REAL
    lpad = jnp.arange(lp, dtype=jnp.int32) % N
    i0_p = jnp.concatenate([i0, lpad]).reshape(NW, LC, CH)
    i1_p = jnp.concatenate([i1, lpad]).reshape(NW, LC, CH)

    x_ext = jnp.zeros((N_EXT, D), jnp.float32).at[:N].set(x)
    zeros_rows = jnp.zeros((ROWS_PER_SUB, D), jnp.float32)
    zeros_1d = jnp.zeros((ROWS_PER_SUB,), jnp.float32)

    wcT = W_conv.T
    bc = b_conv.reshape(1, D)
    w1T = W1.T
    b1r = b1.reshape(1, D // 2)
    w2T = W2.T
    b2r = b2.reshape(1, 1)

    degp = _sc_deg(dst_p, zeros_1d)
    t0 = _tc_scale_x(degp, x_ext)
    u1 = _sc_hop(t0, src_p, dst_p, zeros_rows)
    t1 = _tc_mid(degp, u1)
    u2 = _sc_hop(t1, src_p, dst_p, zeros_rows)
    z = _tc_z(degp, u2, wcT, bc)
    s = _sc_dec(z, i0_p, i1_p)
    r = _tc_mlp(s, w1T, b1r, w2T, b2r)
    return r[:L_REAL, 0]
